# baseline (speedup 1.0000x reference)
import functools

import jax
import jax.numpy as jnp
from jax import lax
from jax.experimental import pallas as pl
from jax.experimental.pallas import tpu as pltpu
from jax.experimental.pallas import tpu_sc as plsc

B = 16384
F = 26
D = 13
VOCAB = 100000
NC = 2
NS = 16
L = 16
NW = NC * NS
R = B // NW
RV = R // L


def _sc_body(xs_hbm, xd_hbm, w_hbm, tab_hbm, out_hbm,
             idx_v, g_v, xd_v, w_v, acc_v, sem):
    cid = lax.axis_index("c")
    sid = lax.axis_index("s")
    wid = sid * NC + cid
    base = wid * R

    pltpu.sync_copy(xs_hbm.at[wid], idx_v)
    pltpu.sync_copy(xd_hbm.at[wid], xd_v)
    pltpu.sync_copy(w_hbm, w_v)

    def add_off(j, _):
        sl = pl.ds(j * L, L)
        off = (j // RV) * VOCAB
        idx_v[sl] = idx_v[sl] + off
        return _

    lax.fori_loop(0, F * RV, add_off, 0, unroll=4)

    gather = pltpu.make_async_copy(tab_hbm.at[idx_v], g_v, sem)
    gather.start()
    gather.wait()

    wrows = [w_v[d] for d in range(D)]

    def reduce_one(j, _):
        sl = pl.ds(j * L, L)
        acc = g_v[sl]
        for f in range(1, F):
            acc = acc + g_v[pl.ds(f * R + j * L, L)]
        for d in range(D):
            acc = acc + xd_v[d, sl] * wrows[d]
        acc_v[sl] = acc
        return _

    lax.fori_loop(0, RV, reduce_one, 0, unroll=2)

    pltpu.sync_copy(acc_v, out_hbm.at[pl.ds(base, R)])


@jax.jit
def _linear_sc(xs_blocks, xd_blocks, w_rep, tab_flat):
    mesh = plsc.VectorSubcoreMesh(core_axis_name="c", subcore_axis_name="s")
    run = pl.kernel(
        _sc_body,
        out_type=jax.ShapeDtypeStruct((B,), jnp.float32),
        mesh=mesh,
        scratch_types=[
            pltpu.VMEM((F * R,), jnp.int32),
            pltpu.VMEM((F * R,), jnp.float32),
            pltpu.VMEM((D, R), jnp.float32),
            pltpu.VMEM((D, L), jnp.float32),
            pltpu.VMEM((R,), jnp.float32),
            pltpu.SemaphoreType.DMA,
        ],
    )
    return run(xs_blocks, xd_blocks, w_rep, tab_flat)


def kernel(X_sparse, X_dense, tables, W_dense):
    xs_blocks = (
        X_sparse.astype(jnp.int32).reshape(NW, R, F).transpose(0, 2, 1)
    ).reshape(NW, F * R)
    xd_blocks = X_dense.reshape(NW, R, D).transpose(0, 2, 1)
    w_rep = jnp.broadcast_to(W_dense.reshape(D, 1), (D, L))
    tab_flat = tables.reshape(F * VOCAB)
    out = _linear_sc(xs_blocks, xd_blocks, w_rep, tab_flat)
    return out.reshape(B, 1)


# speedup vs baseline: 2.2999x; 2.2999x over previous
import functools

import jax
import jax.numpy as jnp
from jax import lax
from jax.experimental import pallas as pl
from jax.experimental.pallas import tpu as pltpu
from jax.experimental.pallas import tpu_sc as plsc

B = 16384
F = 26
D = 13
VOCAB = 100000
VOCAB_P = 100096
NC = 2
NS = 16
L = 16
NW = NC * NS
R = B // NW
RV = R // L


def _sc_body(xs_hbm, xd_hbm, w_hbm, tab_hbm, out_hbm,
             idx_v, g_v, xd_v, w_v, acc_v, sem):
    cid = lax.axis_index("c")
    sid = lax.axis_index("s")
    wid = sid * NC + cid
    base = wid * R

    stage = [
        pltpu.make_async_copy(
            xs_hbm.at[pl.ds(f * B + base, R)], idx_v.at[pl.ds(f * R, R)], sem
        )
        for f in range(F)
    ] + [
        pltpu.make_async_copy(
            xd_hbm.at[pl.ds(d * B + base, R)], xd_v.at[pl.ds(d * R, R)], sem
        )
        for d in range(D)
    ] + [pltpu.make_async_copy(w_hbm, w_v, sem)]
    for c in stage:
        c.start()
    for c in stage:
        c.wait()

    def add_off(j, _):
        sl = pl.ds(j * L, L)
        off = (j // RV) * VOCAB_P
        idx_v[sl] = idx_v[sl] + off
        return _

    lax.fori_loop(0, F * RV, add_off, 0, unroll=4)

    gather = pltpu.make_async_copy(tab_hbm.at[idx_v], g_v, sem)
    gather.start()
    gather.wait()

    wrows = [w_v[d] for d in range(D)]

    def reduce_one(j, _):
        sl = pl.ds(j * L, L)
        acc = g_v[sl]
        for f in range(1, F):
            acc = acc + g_v[pl.ds(f * R + j * L, L)]
        for d in range(D):
            acc = acc + xd_v[pl.ds(d * R + j * L, L)] * wrows[d]
        acc_v[sl] = acc
        return _

    lax.fori_loop(0, RV, reduce_one, 0, unroll=2)

    pltpu.sync_copy(acc_v, out_hbm.at[pl.ds(base, R)])


@jax.jit
def _linear_sc(xs_blocks, xd_blocks, w_rep, tab_flat):
    mesh = plsc.VectorSubcoreMesh(core_axis_name="c", subcore_axis_name="s")
    run = pl.kernel(
        _sc_body,
        out_type=jax.ShapeDtypeStruct((B,), jnp.float32),
        mesh=mesh,
        scratch_types=[
            pltpu.VMEM((F * R,), jnp.int32),
            pltpu.VMEM((F * R,), jnp.float32),
            pltpu.VMEM((D * R,), jnp.float32),
            pltpu.VMEM((D, L), jnp.float32),
            pltpu.VMEM((R,), jnp.float32),
            pltpu.SemaphoreType.DMA,
        ],
    )
    return run(xs_blocks, xd_blocks, w_rep, tab_flat)


def kernel(X_sparse, X_dense, tables, W_dense):
    xs_blocks = X_sparse.astype(jnp.int32).T.reshape(F * B)
    xd_blocks = X_dense.T.reshape(D * B)
    w_rep = jnp.broadcast_to(W_dense.reshape(D, 1), (D, L))
    tab_pad = jnp.pad(
        tables[:, :, 0], ((0, 0), (0, VOCAB_P - VOCAB))
    ).reshape(F * VOCAB_P)
    out = _linear_sc(xs_blocks, xd_blocks, w_rep, tab_pad)
    return out.reshape(B, 1)


# speedup vs baseline: 2.4458x; 1.0635x over previous
import functools

import jax
import jax.numpy as jnp
from jax import lax
from jax.experimental import pallas as pl
from jax.experimental.pallas import tpu as pltpu
from jax.experimental.pallas import tpu_sc as plsc

B = 16384
F = 26
D = 13
VOCAB = 100000
VOCAB_P = 100096
NC = 2
NS = 16
L = 16
NW = NC * NS
R = B // NW
RV = R // L


def _sc_body(xs_hbm, xd_hbm, w_hbm, tab_hbm, out_hbm,
             idx_v, g_v, xd_v, w_v, acc_v, sem):
    cid = lax.axis_index("c")
    sid = lax.axis_index("s")
    wid = sid * NC + cid
    base = wid * R

    stage = [
        pltpu.make_async_copy(
            xs_hbm.at[pl.ds(f * B + base, R)], idx_v.at[pl.ds(f * R, R)], sem
        )
        for f in range(F)
    ] + [
        pltpu.make_async_copy(
            xd_hbm.at[pl.ds(d * B + base, R)], xd_v.at[pl.ds(d * R, R)], sem
        )
        for d in range(D)
    ] + [pltpu.make_async_copy(w_hbm, w_v, sem)]
    for c in stage:
        c.start()
    for c in stage:
        c.wait()

    def add_off(j, _):
        sl = pl.ds(j * L, L)
        off = (j // RV) * VOCAB
        idx_v[sl] = idx_v[sl] + off
        return _

    lax.fori_loop(0, F * RV, add_off, 0, unroll=4)

    gather = pltpu.make_async_copy(tab_hbm.at[idx_v], g_v, sem)
    gather.start()
    gather.wait()

    wrows = [w_v[d] for d in range(D)]

    def reduce_one(j, _):
        sl = pl.ds(j * L, L)
        acc = g_v[sl]
        for f in range(1, F):
            acc = acc + g_v[pl.ds(f * R + j * L, L)]
        for d in range(D):
            acc = acc + xd_v[pl.ds(d * R + j * L, L)] * wrows[d]
        acc_v[sl] = acc
        return _

    lax.fori_loop(0, RV, reduce_one, 0, unroll=2)

    pltpu.sync_copy(acc_v, out_hbm.at[pl.ds(base, R)])


@jax.jit
def _linear_sc(xs_blocks, xd_blocks, w_rep, tab_flat):
    mesh = plsc.VectorSubcoreMesh(core_axis_name="c", subcore_axis_name="s")
    run = pl.kernel(
        _sc_body,
        out_type=jax.ShapeDtypeStruct((B,), jnp.float32),
        mesh=mesh,
        scratch_types=[
            pltpu.VMEM((F * R,), jnp.int32),
            pltpu.VMEM((F * R,), jnp.float32),
            pltpu.VMEM((D * R,), jnp.float32),
            pltpu.VMEM((D, L), jnp.float32),
            pltpu.VMEM((R,), jnp.float32),
            pltpu.SemaphoreType.DMA,
        ],
    )
    return run(xs_blocks, xd_blocks, w_rep, tab_flat)


def kernel(X_sparse, X_dense, tables, W_dense):
    xs_blocks = X_sparse.astype(jnp.int32).T.reshape(F * B)
    xd_blocks = X_dense.T.reshape(D * B)
    w_rep = jnp.broadcast_to(W_dense.reshape(D, 1), (D, L))
    tab2d = lax.optimization_barrier(tables[:, :, 0])
    tab_pad = tab2d.reshape(F * VOCAB)
    out = _linear_sc(xs_blocks, xd_blocks, w_rep, tab_pad)
    return out.reshape(B, 1)
